# baseline (speedup 1.0000x reference)
import functools

import jax
import jax.numpy as jnp
from jax import lax
from jax.experimental import pallas as pl
from jax.experimental.pallas import tpu as pltpu
from jax.experimental.pallas import tpu_sc as plsc

H = 8
C = 16
D = 128
TW = 144
N_PAPER, N_AUTHOR, N_SUBJECT = 50000, 40000, 10000
E_PA, E_PS = 256000, 32000

_HIGH = lax.Precision.HIGHEST


def _att_to_r16(att):
    a = att.reshape(D)
    sel = (jnp.arange(C)[None, :] == (jnp.arange(D) // C)[:, None])
    return a[:, None] * sel.astype(jnp.float32)


def _prep_src(x, W, R16):
    n = x.shape[0]
    bn = 2000

    def body(x_ref, w_ref, r_ref, o_ref):
        h = jnp.dot(x_ref[...], w_ref[...], precision=_HIGH,
                    preferred_element_type=jnp.float32)
        es = jnp.dot(h, r_ref[...], precision=_HIGH,
                     preferred_element_type=jnp.float32)
        o_ref[...] = jnp.concatenate([h, es], axis=-1)

    return pl.pallas_call(
        body,
        grid=(n // bn,),
        in_specs=[
            pl.BlockSpec((bn, D), lambda i: (i, 0)),
            pl.BlockSpec((D, D), lambda i: (0, 0)),
            pl.BlockSpec((D, C), lambda i: (0, 0)),
        ],
        out_specs=pl.BlockSpec((bn, TW), lambda i: (i, 0)),
        out_shape=jax.ShapeDtypeStruct((n, TW), jnp.float32),
    )(x, W, R16)


def _prep_dst(x, W, R16):
    n = x.shape[0]
    bn = 2000

    def body(x_ref, w_ref, r_ref, o_ref):
        h = jnp.dot(x_ref[...], w_ref[...], precision=_HIGH,
                    preferred_element_type=jnp.float32)
        o_ref[...] = jnp.dot(h, r_ref[...], precision=_HIGH,
                             preferred_element_type=jnp.float32)

    return pl.pallas_call(
        body,
        grid=(n // bn,),
        in_specs=[
            pl.BlockSpec((bn, D), lambda i: (i, 0)),
            pl.BlockSpec((D, D), lambda i: (0, 0)),
            pl.BlockSpec((D, C), lambda i: (0, 0)),
        ],
        out_specs=pl.BlockSpec((bn, C), lambda i: (i, 0)),
        out_shape=jax.ShapeDtypeStruct((n, C), jnp.float32),
    )(x, W, R16)


def _make_edge_kernel(n_dst, n_edges, k_chunks, chunk_n):
    B = 128
    nblk = n_edges // B
    npad = k_chunks * chunk_n
    acc_rows = chunk_n + 128
    rpt = chunk_n // 16
    rpt2 = acc_rows // 16
    zf, zr = rpt2 // B, rpt2 % B
    mesh = plsc.VectorSubcoreMesh(core_axis_name="c", subcore_axis_name="s")

    @functools.partial(
        pl.kernel,
        out_type=jax.ShapeDtypeStruct((2, npad, TW), jnp.float32),
        mesh=mesh,
        compiler_params=pltpu.CompilerParams(use_tc_tiling_on_sc=False),
        scratch_types=[
            pltpu.VMEM((B,), jnp.int32),
            pltpu.VMEM((B,), jnp.int32),
            pltpu.VMEM((B,), jnp.int32),
            pltpu.VMEM((B,), jnp.int32),
            pltpu.VMEM((B,), jnp.int32),
            pltpu.VMEM((2 * B,), jnp.int32),
            pltpu.VMEM((2 * B,), jnp.int32),
            pltpu.VMEM((16,), jnp.int32),
            pltpu.VMEM((B, TW), jnp.float32),
            pltpu.VMEM((B, C), jnp.float32),
            pltpu.VMEM_SHARED((acc_rows, TW), jnp.float32),
        ],
    )
    def edge_kernel(stab, edtab, sid_hbm, did_hbm, out_hbm,
                    sid_v, did_v, l_v, inc_v, lidx_v, stg_s, stg_l, cnt_v,
                    srow_v, edrow_v, acc):
        cid = lax.axis_index("c")
        tid = lax.axis_index("s")
        wid = tid * 2 + cid

        @pl.loop(0, 2 * B, step=16)
        def _(i):
            stg_s[pl.ds(i, 16)] = jnp.zeros((16,), jnp.int32)
            stg_l[pl.ds(i, 16)] = jnp.zeros((16,), jnp.int32)

        def flush(base, mcnt):
            @pl.loop(0, B, step=16)
            def _(i):
                sv = stg_s[pl.ds(i, 16)]
                lv = stg_l[pl.ds(i, 16)]
                sid_v[pl.ds(i, 16)] = sv
                dd = lv + base
                did_v[pl.ds(i, 16)] = jnp.clip(dd, 0, n_dst - 1)
                if mcnt is None:
                    lidx_v[pl.ds(i, 16)] = lv
                else:
                    io = lax.iota(jnp.int32, 16) + i
                    lidx_v[pl.ds(i, 16)] = jnp.where(io < mcnt, lv, chunk_n)

            pltpu.sync_copy(stab.at[sid_v], srow_v)
            pltpu.sync_copy(edtab.at[did_v], edrow_v)

            @pl.loop(0, B)
            def _(e):
                es = srow_v[e, pl.ds(D, C)]
                ed = edrow_v[e, pl.ds(0, C)]
                a = es + ed
                a = jnp.where(a >= 0, a, a * 0.2)
                w = jnp.exp(a)
                srow_v[e, pl.ds(D, C)] = w
                for h in range(H):
                    srow_v[e, pl.ds(h * C, C)] = (
                        srow_v[e, pl.ds(h * C, C)] * w[h])

            pltpu.sync_copy(srow_v, acc.at[lidx_v], add=True)

        for k in range(k_chunks):
            base = k * chunk_n

            @pl.loop(0, B)
            def _(r):
                for j in range(TW // C):
                    srow_v[r, pl.ds(j * C, C)] = jnp.zeros((C,), jnp.float32)

            z0 = tid * rpt2
            for jz in range(zf):
                pltpu.sync_copy(srow_v, acc.at[pl.ds(z0 + jz * B, B)])
            if zr:
                pltpu.sync_copy(srow_v.at[pl.ds(0, zr)],
                                acc.at[pl.ds(z0 + zf * B, zr)])
            cnt_v[pl.ds(0, 16)] = jnp.zeros((16,), jnp.int32)
            plsc.subcore_barrier()

            @pl.loop(wid, nblk, step=32)
            def _(blk):
                off = blk * B
                pltpu.sync_copy(sid_hbm.at[pl.ds(off, B)], sid_v)
                pltpu.sync_copy(did_hbm.at[pl.ds(off, B)], l_v)

                @pl.loop(0, B, step=16)
                def _(i):
                    l = l_v[pl.ds(i, 16)] - base
                    inb = (l >= 0) & (l < chunk_n)
                    l_v[pl.ds(i, 16)] = l
                    inc_v[pl.ds(i, 16)] = jnp.where(inb, 1, 0)

                @pl.loop(0, B)
                def _(e):
                    c = cnt_v[pl.ds(0, 1)][0]
                    stg_s[pl.ds(c, 1)] = sid_v[pl.ds(e, 1)]
                    stg_l[pl.ds(c, 1)] = l_v[pl.ds(e, 1)]
                    cnt_v[pl.ds(0, 1)] = (c + inc_v[pl.ds(e, 1)][0]).reshape(1)

                @pl.when(cnt_v[pl.ds(0, 1)][0] >= B)
                def _():
                    flush(base, None)

                    @pl.loop(0, B, step=16)
                    def _(i):
                        stg_s[pl.ds(i, 16)] = stg_s[pl.ds(B + i, 16)]
                        stg_l[pl.ds(i, 16)] = stg_l[pl.ds(B + i, 16)]
                    cnt_v[pl.ds(0, 1)] = (
                        cnt_v[pl.ds(0, 1)][0] - B).reshape(1)

            @pl.when(cnt_v[pl.ds(0, 1)][0] > 0)
            def _():
                flush(base, cnt_v[pl.ds(0, 1)][0])

            plsc.subcore_barrier()
            d0 = tid * rpt
            pltpu.sync_copy(acc.at[pl.ds(d0, rpt)],
                            out_hbm.at[cid, pl.ds(base + d0, rpt)])
            plsc.subcore_barrier()

    return edge_kernel


def _head_expand():
    return (jnp.arange(H)[:, None] == (jnp.arange(D) // C)[None, :]).astype(
        jnp.float32)


def _combine1(part, b, n_dst):
    bn = 2000
    E8 = _head_expand()

    def body(p_ref, b_ref, e_ref, o_ref):
        p = p_ref[...]
        num = p[0, :, 0:D] + p[1, :, 0:D]
        den = p[0, :, D:D + H] + p[1, :, D:D + H]
        rec = 1.0 / (den + 1e-16)
        rex = jnp.dot(rec, e_ref[...], precision=_HIGH,
                      preferred_element_type=jnp.float32)
        o_ref[...] = num * rex + b_ref[...]

    return pl.pallas_call(
        body,
        grid=(n_dst // bn,),
        in_specs=[
            pl.BlockSpec((2, bn, TW), lambda i: (0, i, 0)),
            pl.BlockSpec((1, D), lambda i: (0, 0)),
            pl.BlockSpec((H, D), lambda i: (0, 0)),
        ],
        out_specs=pl.BlockSpec((bn, D), lambda i: (i, 0)),
        out_shape=jax.ShapeDtypeStruct((n_dst, D), jnp.float32),
    )(part, b.reshape(1, D), E8)


def _combine2(part_a, b_a, part_b, b_b, n_dst):
    bn = 2000
    E8 = _head_expand()

    def body(pa_ref, pb_ref, ba_ref, bb_ref, e_ref, o_ref):
        outs = []
        for p_ref, bias in ((pa_ref, ba_ref), (pb_ref, bb_ref)):
            p = p_ref[...]
            num = p[0, :, 0:D] + p[1, :, 0:D]
            den = p[0, :, D:D + H] + p[1, :, D:D + H]
            rec = 1.0 / (den + 1e-16)
            rex = jnp.dot(rec, e_ref[...], precision=_HIGH,
                          preferred_element_type=jnp.float32)
            outs.append(num * rex + bias[...])
        o_ref[...] = (outs[0] + outs[1]) * 0.5

    return pl.pallas_call(
        body,
        grid=(n_dst // bn,),
        in_specs=[
            pl.BlockSpec((2, bn, TW), lambda i: (0, i, 0)),
            pl.BlockSpec((2, bn, TW), lambda i: (0, i, 0)),
            pl.BlockSpec((1, D), lambda i: (0, 0)),
            pl.BlockSpec((1, D), lambda i: (0, 0)),
            pl.BlockSpec((H, D), lambda i: (0, 0)),
        ],
        out_specs=pl.BlockSpec((bn, D), lambda i: (i, 0)),
        out_shape=jax.ShapeDtypeStruct((n_dst, D), jnp.float32),
    )(part_a, part_b, b_a.reshape(1, D), b_b.reshape(1, D), E8)


_CFG = {
    "pa": (N_AUTHOR, E_PA, 4, 10112),
    "ap": (N_PAPER, E_PA, 5, 10112),
    "ps": (N_SUBJECT, E_PS, 1, 10112),
    "sp": (N_PAPER, E_PS, 5, 10112),
}
_EDGE_KERNELS = {
    name: _make_edge_kernel(n_dst, n_edges, k, cn)
    for name, (n_dst, n_edges, k, cn) in _CFG.items()
}


def _relation(name, x_src, x_dst, src, dst, W, a_src, a_dst):
    stab = _prep_src(x_src, W, _att_to_r16(a_src))
    edtab = _prep_dst(x_dst, W, _att_to_r16(a_dst))
    return _EDGE_KERNELS[name](stab, edtab, src, dst)


def kernel(x_paper, x_author, x_subject, src_pa, dst_pa, src_ap, dst_ap,
           src_ps, dst_ps, src_sp, dst_sp, W_pa, att_src_pa, att_dst_pa, b_pa,
           W_ap, att_src_ap, att_dst_ap, b_ap, W_ps, att_src_ps, att_dst_ps,
           b_ps, W_sp, att_src_sp, att_dst_sp, b_sp):
    part_pa = _relation("pa", x_paper, x_author, src_pa, dst_pa,
                        W_pa, att_src_pa, att_dst_pa)
    part_ap = _relation("ap", x_author, x_paper, src_ap, dst_ap,
                        W_ap, att_src_ap, att_dst_ap)
    part_ps = _relation("ps", x_paper, x_subject, src_ps, dst_ps,
                        W_ps, att_src_ps, att_dst_ps)
    part_sp = _relation("sp", x_subject, x_paper, src_sp, dst_sp,
                        W_sp, att_src_sp, att_dst_sp)

    out_author = _combine1(part_pa, b_pa, N_AUTHOR)
    out_subject = _combine1(part_ps, b_ps, N_SUBJECT)
    out_paper = _combine2(part_ap, b_ap, part_sp, b_sp, N_PAPER)
    out = jnp.concatenate([out_paper, out_author, out_subject], axis=0)
    return out[None, :, :]


# speedup vs baseline: 1.0020x; 1.0020x over previous
import functools

import jax
import jax.numpy as jnp
from jax import lax
from jax.experimental import pallas as pl
from jax.experimental.pallas import tpu as pltpu
from jax.experimental.pallas import tpu_sc as plsc

H = 8
C = 16
D = 128
TW = 144
N_PAPER, N_AUTHOR, N_SUBJECT = 50000, 40000, 10000
E_PA, E_PS = 256000, 32000

_HIGH = lax.Precision.HIGHEST


def _att_to_r16(att):
    a = att.reshape(D)
    sel = (jnp.arange(C)[None, :] == (jnp.arange(D) // C)[:, None])
    return a[:, None] * sel.astype(jnp.float32)


def _prep_src(x, W, R16):
    n = x.shape[0]
    bn = 2000

    def body(x_ref, w_ref, r_ref, o_ref):
        h = jnp.dot(x_ref[...], w_ref[...], precision=_HIGH,
                    preferred_element_type=jnp.float32)
        es = jnp.dot(h, r_ref[...], precision=_HIGH,
                     preferred_element_type=jnp.float32)
        o_ref[...] = jnp.concatenate([h, es], axis=-1)

    return pl.pallas_call(
        body,
        grid=(n // bn,),
        in_specs=[
            pl.BlockSpec((bn, D), lambda i: (i, 0)),
            pl.BlockSpec((D, D), lambda i: (0, 0)),
            pl.BlockSpec((D, C), lambda i: (0, 0)),
        ],
        out_specs=pl.BlockSpec((bn, TW), lambda i: (i, 0)),
        out_shape=jax.ShapeDtypeStruct((n, TW), jnp.float32),
    )(x, W, R16)


def _prep_dst(x, W, R16):
    n = x.shape[0]
    bn = 2000

    def body(x_ref, w_ref, r_ref, o_ref):
        h = jnp.dot(x_ref[...], w_ref[...], precision=_HIGH,
                    preferred_element_type=jnp.float32)
        o_ref[...] = jnp.dot(h, r_ref[...], precision=_HIGH,
                             preferred_element_type=jnp.float32)

    return pl.pallas_call(
        body,
        grid=(n // bn,),
        in_specs=[
            pl.BlockSpec((bn, D), lambda i: (i, 0)),
            pl.BlockSpec((D, D), lambda i: (0, 0)),
            pl.BlockSpec((D, C), lambda i: (0, 0)),
        ],
        out_specs=pl.BlockSpec((bn, C), lambda i: (i, 0)),
        out_shape=jax.ShapeDtypeStruct((n, C), jnp.float32),
    )(x, W, R16)


def _make_edge_kernel(n_dst, n_edges, k_chunks, chunk_n):
    B = 64
    nblk = n_edges // B
    npad = k_chunks * chunk_n
    acc_rows = chunk_n + 128
    rpt = chunk_n // 16
    rpt2 = acc_rows // 16
    zf, zr = rpt2 // B, rpt2 % B
    mesh = plsc.VectorSubcoreMesh(core_axis_name="c", subcore_axis_name="s")

    @functools.partial(
        pl.kernel,
        out_type=jax.ShapeDtypeStruct((2, npad, TW), jnp.float32),
        mesh=mesh,
        compiler_params=pltpu.CompilerParams(use_tc_tiling_on_sc=False),
        scratch_types=[
            pltpu.VMEM((B,), jnp.int32),
            pltpu.VMEM((B,), jnp.int32),
            pltpu.VMEM((B,), jnp.int32),
            pltpu.VMEM((B, TW), jnp.float32),
            pltpu.VMEM((B, C), jnp.float32),
            pltpu.VMEM_SHARED((acc_rows, TW), jnp.float32),
        ],
    )
    def edge_kernel(stab, edtab, sid_hbm, did_hbm, out_hbm,
                    sid_v, did_v, lidx_v, srow_v, edrow_v, acc):
        cid = lax.axis_index("c")
        tid = lax.axis_index("s")
        wid = tid * 2 + cid

        for k in range(k_chunks):
            base = k * chunk_n

            @pl.loop(0, B)
            def _(r):
                for j in range(TW // C):
                    srow_v[r, pl.ds(j * C, C)] = jnp.zeros((C,), jnp.float32)

            z0 = tid * rpt2
            for jz in range(zf):
                pltpu.sync_copy(srow_v, acc.at[pl.ds(z0 + jz * B, B)])
            if zr:
                pltpu.sync_copy(srow_v.at[pl.ds(0, zr)],
                                acc.at[pl.ds(z0 + zf * B, zr)])
            plsc.subcore_barrier()

            @pl.loop(wid, nblk, step=32)
            def _(blk):
                off = blk * B
                pltpu.sync_copy(sid_hbm.at[pl.ds(off, B)], sid_v)
                pltpu.sync_copy(did_hbm.at[pl.ds(off, B)], did_v)

                @pl.loop(0, B, step=16)
                def _(i):
                    l = did_v[pl.ds(i, 16)] - base
                    inb = (l >= 0) & (l < chunk_n)
                    lidx_v[pl.ds(i, 16)] = jnp.where(inb, l, chunk_n)

                pltpu.sync_copy(stab.at[sid_v], srow_v)
                pltpu.sync_copy(edtab.at[did_v], edrow_v)

                @pl.loop(0, B)
                def _(e):
                    es = srow_v[e, pl.ds(D, C)]
                    ed = edrow_v[e, pl.ds(0, C)]
                    a = es + ed
                    a = jnp.where(a >= 0, a, a * 0.2)
                    w = jnp.exp(a)
                    srow_v[e, pl.ds(D, C)] = w
                    for h in range(H):
                        srow_v[e, pl.ds(h * C, C)] = (
                            srow_v[e, pl.ds(h * C, C)] * w[h])

                pltpu.sync_copy(srow_v, acc.at[lidx_v], add=True)

            plsc.subcore_barrier()
            d0 = tid * rpt
            pltpu.sync_copy(acc.at[pl.ds(d0, rpt)],
                            out_hbm.at[cid, pl.ds(base + d0, rpt)])
            plsc.subcore_barrier()

    return edge_kernel


def _head_expand():
    return (jnp.arange(H)[:, None] == (jnp.arange(D) // C)[None, :]).astype(
        jnp.float32)


def _combine1(part, b, n_dst):
    bn = 2000
    E8 = _head_expand()

    def body(p_ref, b_ref, e_ref, o_ref):
        p = p_ref[...]
        num = p[0, :, 0:D] + p[1, :, 0:D]
        den = p[0, :, D:D + H] + p[1, :, D:D + H]
        rec = 1.0 / (den + 1e-16)
        rex = jnp.dot(rec, e_ref[...], precision=_HIGH,
                      preferred_element_type=jnp.float32)
        o_ref[...] = num * rex + b_ref[...]

    return pl.pallas_call(
        body,
        grid=(n_dst // bn,),
        in_specs=[
            pl.BlockSpec((2, bn, TW), lambda i: (0, i, 0)),
            pl.BlockSpec((1, D), lambda i: (0, 0)),
            pl.BlockSpec((H, D), lambda i: (0, 0)),
        ],
        out_specs=pl.BlockSpec((bn, D), lambda i: (i, 0)),
        out_shape=jax.ShapeDtypeStruct((n_dst, D), jnp.float32),
    )(part, b.reshape(1, D), E8)


def _combine2(part_a, b_a, part_b, b_b, n_dst):
    bn = 2000
    E8 = _head_expand()

    def body(pa_ref, pb_ref, ba_ref, bb_ref, e_ref, o_ref):
        outs = []
        for p_ref, bias in ((pa_ref, ba_ref), (pb_ref, bb_ref)):
            p = p_ref[...]
            num = p[0, :, 0:D] + p[1, :, 0:D]
            den = p[0, :, D:D + H] + p[1, :, D:D + H]
            rec = 1.0 / (den + 1e-16)
            rex = jnp.dot(rec, e_ref[...], precision=_HIGH,
                          preferred_element_type=jnp.float32)
            outs.append(num * rex + bias[...])
        o_ref[...] = (outs[0] + outs[1]) * 0.5

    return pl.pallas_call(
        body,
        grid=(n_dst // bn,),
        in_specs=[
            pl.BlockSpec((2, bn, TW), lambda i: (0, i, 0)),
            pl.BlockSpec((2, bn, TW), lambda i: (0, i, 0)),
            pl.BlockSpec((1, D), lambda i: (0, 0)),
            pl.BlockSpec((1, D), lambda i: (0, 0)),
            pl.BlockSpec((H, D), lambda i: (0, 0)),
        ],
        out_specs=pl.BlockSpec((bn, D), lambda i: (i, 0)),
        out_shape=jax.ShapeDtypeStruct((n_dst, D), jnp.float32),
    )(part_a, part_b, b_a.reshape(1, D), b_b.reshape(1, D), E8)


_CFG = {
    "pa": (N_AUTHOR, E_PA, 4, 12544),
    "ap": (N_PAPER, E_PA, 4, 12544),
    "ps": (N_SUBJECT, E_PS, 1, 12544),
    "sp": (N_PAPER, E_PS, 4, 12544),
}
_EDGE_KERNELS = {
    name: _make_edge_kernel(n_dst, n_edges, k, cn)
    for name, (n_dst, n_edges, k, cn) in _CFG.items()
}


def _relation(name, x_src, x_dst, src, dst, W, a_src, a_dst):
    stab = _prep_src(x_src, W, _att_to_r16(a_src))
    edtab = _prep_dst(x_dst, W, _att_to_r16(a_dst))
    return _EDGE_KERNELS[name](stab, edtab, src, dst)


def kernel(x_paper, x_author, x_subject, src_pa, dst_pa, src_ap, dst_ap,
           src_ps, dst_ps, src_sp, dst_sp, W_pa, att_src_pa, att_dst_pa, b_pa,
           W_ap, att_src_ap, att_dst_ap, b_ap, W_ps, att_src_ps, att_dst_ps,
           b_ps, W_sp, att_src_sp, att_dst_sp, b_sp):
    part_pa = _relation("pa", x_paper, x_author, src_pa, dst_pa,
                        W_pa, att_src_pa, att_dst_pa)
    part_ap = _relation("ap", x_author, x_paper, src_ap, dst_ap,
                        W_ap, att_src_ap, att_dst_ap)
    part_ps = _relation("ps", x_paper, x_subject, src_ps, dst_ps,
                        W_ps, att_src_ps, att_dst_ps)
    part_sp = _relation("sp", x_subject, x_paper, src_sp, dst_sp,
                        W_sp, att_src_sp, att_dst_sp)

    out_author = _combine1(part_pa, b_pa, N_AUTHOR)
    out_subject = _combine1(part_ps, b_ps, N_SUBJECT)
    out_paper = _combine2(part_ap, b_ap, part_sp, b_sp, N_PAPER)
    out = jnp.concatenate([out_paper, out_author, out_subject], axis=0)
    return out[None, :, :]


# speedup vs baseline: 1.1350x; 1.1327x over previous
import functools

import jax
import jax.numpy as jnp
from jax import lax
from jax.experimental import pallas as pl
from jax.experimental.pallas import tpu as pltpu
from jax.experimental.pallas import tpu_sc as plsc

H = 8
C = 16
D = 128
TW = 144
N_PAPER, N_AUTHOR, N_SUBJECT = 50000, 40000, 10000
E_PA, E_PS = 256000, 32000

_HIGH = lax.Precision.HIGHEST


def _att_to_r16(att):
    a = att.reshape(D)
    sel = (jnp.arange(C)[None, :] == (jnp.arange(D) // C)[:, None])
    return a[:, None] * sel.astype(jnp.float32)


def _prep_src(x, W, R16):
    n = x.shape[0]
    bn = 2000

    def body(x_ref, w_ref, r_ref, o_ref):
        h = jnp.dot(x_ref[...], w_ref[...], precision=_HIGH,
                    preferred_element_type=jnp.float32)
        es = jnp.dot(h, r_ref[...], precision=_HIGH,
                     preferred_element_type=jnp.float32)
        o_ref[...] = jnp.concatenate([h, es], axis=-1)

    return pl.pallas_call(
        body,
        grid=(n // bn,),
        in_specs=[
            pl.BlockSpec((bn, D), lambda i: (i, 0)),
            pl.BlockSpec((D, D), lambda i: (0, 0)),
            pl.BlockSpec((D, C), lambda i: (0, 0)),
        ],
        out_specs=pl.BlockSpec((bn, TW), lambda i: (i, 0)),
        out_shape=jax.ShapeDtypeStruct((n, TW), jnp.float32),
    )(x, W, R16)


def _prep_dst(x, W, R16):
    n = x.shape[0]
    bn = 2000

    def body(x_ref, w_ref, r_ref, o_ref):
        h = jnp.dot(x_ref[...], w_ref[...], precision=_HIGH,
                    preferred_element_type=jnp.float32)
        o_ref[...] = jnp.dot(h, r_ref[...], precision=_HIGH,
                             preferred_element_type=jnp.float32)

    return pl.pallas_call(
        body,
        grid=(n // bn,),
        in_specs=[
            pl.BlockSpec((bn, D), lambda i: (i, 0)),
            pl.BlockSpec((D, D), lambda i: (0, 0)),
            pl.BlockSpec((D, C), lambda i: (0, 0)),
        ],
        out_specs=pl.BlockSpec((bn, C), lambda i: (i, 0)),
        out_shape=jax.ShapeDtypeStruct((n, C), jnp.float32),
    )(x, W, R16)


def _make_edge_kernel(n_dst, n_edges, k_chunks, chunk_n):
    B = 128
    nblk = n_edges // B
    npad = k_chunks * chunk_n
    acc_rows = chunk_n + 128
    rpt = chunk_n // 16
    rpt2 = acc_rows // 16
    zf, zr = rpt2 // B, rpt2 % B
    mesh = plsc.VectorSubcoreMesh(core_axis_name="c", subcore_axis_name="s")

    @functools.partial(
        pl.kernel,
        out_type=jax.ShapeDtypeStruct((2, npad, TW), jnp.float32),
        mesh=mesh,
        compiler_params=pltpu.CompilerParams(use_tc_tiling_on_sc=False),
        scratch_types=[
            pltpu.VMEM((B,), jnp.int32),
            pltpu.VMEM((B,), jnp.int32),
            pltpu.VMEM((B,), jnp.int32),
            pltpu.VMEM((B, TW), jnp.float32),
            pltpu.VMEM((B, C), jnp.float32),
            pltpu.VMEM_SHARED((acc_rows, TW), jnp.float32),
        ],
    )
    def edge_kernel(stab, edtab, sid_hbm, did_hbm, out_hbm,
                    sid_v, did_v, lidx_v, srow_v, edrow_v, acc):
        cid = lax.axis_index("c")
        tid = lax.axis_index("s")
        wid = tid * 2 + cid

        for k in range(k_chunks):
            base = k * chunk_n

            @pl.loop(0, B)
            def _(r):
                for j in range(TW // C):
                    srow_v[r, pl.ds(j * C, C)] = jnp.zeros((C,), jnp.float32)

            z0 = tid * rpt2
            for jz in range(zf):
                pltpu.sync_copy(srow_v, acc.at[pl.ds(z0 + jz * B, B)])
            if zr:
                pltpu.sync_copy(srow_v.at[pl.ds(0, zr)],
                                acc.at[pl.ds(z0 + zf * B, zr)])
            plsc.subcore_barrier()

            @pl.loop(wid, nblk, step=32)
            def _(blk):
                off = blk * B
                pltpu.sync_copy(sid_hbm.at[pl.ds(off, B)], sid_v)
                pltpu.sync_copy(did_hbm.at[pl.ds(off, B)], did_v)

                @pl.loop(0, B, step=16)
                def _(i):
                    l = did_v[pl.ds(i, 16)] - base
                    inb = (l >= 0) & (l < chunk_n)
                    lidx_v[pl.ds(i, 16)] = jnp.where(inb, l, chunk_n)

                pltpu.sync_copy(stab.at[sid_v], srow_v)
                pltpu.sync_copy(edtab.at[did_v], edrow_v)

                @pl.loop(0, B)
                def _(e):
                    es = srow_v[e, pl.ds(D, C)]
                    ed = edrow_v[e, pl.ds(0, C)]
                    a = es + ed
                    a = jnp.where(a >= 0, a, a * 0.2)
                    w = jnp.exp(a)
                    srow_v[e, pl.ds(D, C)] = w
                    for h in range(H):
                        srow_v[e, pl.ds(h * C, C)] = (
                            srow_v[e, pl.ds(h * C, C)] * w[h])

                pltpu.sync_copy(srow_v, acc.at[lidx_v], add=True)

            plsc.subcore_barrier()
            d0 = tid * rpt
            pltpu.sync_copy(acc.at[pl.ds(d0, rpt)],
                            out_hbm.at[cid, pl.ds(base + d0, rpt)])
            plsc.subcore_barrier()

    return edge_kernel


def _head_expand():
    return (jnp.arange(H)[:, None] == (jnp.arange(D) // C)[None, :]).astype(
        jnp.float32)


def _combine1(part, b, n_dst):
    bn = 2000
    E8 = _head_expand()

    def body(p_ref, b_ref, e_ref, o_ref):
        p = p_ref[...]
        num = p[0, :, 0:D] + p[1, :, 0:D]
        den = p[0, :, D:D + H] + p[1, :, D:D + H]
        rec = 1.0 / (den + 1e-16)
        rex = jnp.dot(rec, e_ref[...], precision=_HIGH,
                      preferred_element_type=jnp.float32)
        o_ref[...] = num * rex + b_ref[...]

    return pl.pallas_call(
        body,
        grid=(n_dst // bn,),
        in_specs=[
            pl.BlockSpec((2, bn, TW), lambda i: (0, i, 0)),
            pl.BlockSpec((1, D), lambda i: (0, 0)),
            pl.BlockSpec((H, D), lambda i: (0, 0)),
        ],
        out_specs=pl.BlockSpec((bn, D), lambda i: (i, 0)),
        out_shape=jax.ShapeDtypeStruct((n_dst, D), jnp.float32),
    )(part, b.reshape(1, D), E8)


def _combine2(part_a, b_a, part_b, b_b, n_dst):
    bn = 2000
    E8 = _head_expand()

    def body(pa_ref, pb_ref, ba_ref, bb_ref, e_ref, o_ref):
        outs = []
        for p_ref, bias in ((pa_ref, ba_ref), (pb_ref, bb_ref)):
            p = p_ref[...]
            num = p[0, :, 0:D] + p[1, :, 0:D]
            den = p[0, :, D:D + H] + p[1, :, D:D + H]
            rec = 1.0 / (den + 1e-16)
            rex = jnp.dot(rec, e_ref[...], precision=_HIGH,
                          preferred_element_type=jnp.float32)
            outs.append(num * rex + bias[...])
        o_ref[...] = (outs[0] + outs[1]) * 0.5

    return pl.pallas_call(
        body,
        grid=(n_dst // bn,),
        in_specs=[
            pl.BlockSpec((2, bn, TW), lambda i: (0, i, 0)),
            pl.BlockSpec((2, bn, TW), lambda i: (0, i, 0)),
            pl.BlockSpec((1, D), lambda i: (0, 0)),
            pl.BlockSpec((1, D), lambda i: (0, 0)),
            pl.BlockSpec((H, D), lambda i: (0, 0)),
        ],
        out_specs=pl.BlockSpec((bn, D), lambda i: (i, 0)),
        out_shape=jax.ShapeDtypeStruct((n_dst, D), jnp.float32),
    )(part_a, part_b, b_a.reshape(1, D), b_b.reshape(1, D), E8)


_CFG = {
    "pa": (N_AUTHOR, E_PA, 4, 10112),
    "ap": (N_PAPER, E_PA, 5, 10112),
    "ps": (N_SUBJECT, E_PS, 1, 10112),
    "sp": (N_PAPER, E_PS, 5, 10112),
}
_EDGE_KERNELS = {
    name: _make_edge_kernel(n_dst, n_edges, k, cn)
    for name, (n_dst, n_edges, k, cn) in _CFG.items()
}


def _relation(name, x_src, x_dst, src, dst, W, a_src, a_dst):
    stab = _prep_src(x_src, W, _att_to_r16(a_src))
    edtab = _prep_dst(x_dst, W, _att_to_r16(a_dst))
    return _EDGE_KERNELS[name](stab, edtab, src, dst)


def kernel(x_paper, x_author, x_subject, src_pa, dst_pa, src_ap, dst_ap,
           src_ps, dst_ps, src_sp, dst_sp, W_pa, att_src_pa, att_dst_pa, b_pa,
           W_ap, att_src_ap, att_dst_ap, b_ap, W_ps, att_src_ps, att_dst_ps,
           b_ps, W_sp, att_src_sp, att_dst_sp, b_sp):
    part_pa = _relation("pa", x_paper, x_author, src_pa, dst_pa,
                        W_pa, att_src_pa, att_dst_pa)
    part_ap = _relation("ap", x_author, x_paper, src_ap, dst_ap,
                        W_ap, att_src_ap, att_dst_ap)
    part_ps = _relation("ps", x_paper, x_subject, src_ps, dst_ps,
                        W_ps, att_src_ps, att_dst_ps)
    part_sp = _relation("sp", x_subject, x_paper, src_sp, dst_sp,
                        W_sp, att_src_sp, att_dst_sp)

    out_author = _combine1(part_pa, b_pa, N_AUTHOR)
    out_subject = _combine1(part_ps, b_ps, N_SUBJECT)
    out_paper = _combine2(part_ap, b_ap, part_sp, b_sp, N_PAPER)
    out = jnp.concatenate([out_paper, out_author, out_subject], axis=0)
    return out[None, :, :]
